# baseline (speedup 1.0000x reference)
import functools

import jax
import jax.numpy as jnp
from jax import lax
from jax.experimental import pallas as pl
from jax.experimental.pallas import tpu as pltpu
from jax.experimental.pallas import tpu_sc as plsc

DIM = 32
NUM_CORES = 2
NUM_SUBCORES = 16
NW = NUM_CORES * NUM_SUBCORES

S = 200
N = 16384
NB = N // 128
CHUNK = 256
K = N // CHUNK
N_CHUNKS = (S * K) // NW


def _sc_gather(x3, table):
    mesh = plsc.VectorSubcoreMesh(core_axis_name="c", subcore_axis_name="s")

    @functools.partial(
        pl.kernel,
        mesh=mesh,
        out_type=jax.ShapeDtypeStruct((S, 4, NB, 8, 128), jnp.float32),
        scratch_types=[
            pltpu.VMEM((2, CHUNK), jnp.int32),
            pltpu.VMEM((2, CHUNK, DIM), jnp.float32),
            pltpu.VMEM((2, 4, 8, 128), jnp.float32),
            pltpu.SemaphoreType.DMA((2,)),
            pltpu.SemaphoreType.DMA((2,)),
            pltpu.SemaphoreType.DMA((2,)),
        ],
        compiler_params=pltpu.CompilerParams(
            use_tc_tiling_on_sc=False, needs_layout_passes=False),
    )
    def k(x_hbm, table_hbm, out_hbm, idx_v, rows_v, out_v,
          sem_i, sem_g, sem_o):
        wid = lax.axis_index("s") * NUM_CORES + lax.axis_index("c")
        iota = lax.iota(jnp.int32, 16)
        base = wid * N_CHUNKS

        def idx_copy(i, sl):
            g = base + i
            return pltpu.make_async_copy(
                x_hbm.at[g // K, g % K], idx_v.at[sl], sem_i.at[sl])

        def gather(sl):
            return pltpu.make_async_copy(
                table_hbm.at[idx_v.at[sl]], rows_v.at[sl], sem_g.at[sl])

        def store(i, ob, dh):
            g = base + i
            nh = (g % K) * 2 + ob
            return pltpu.make_async_copy(
                out_v.at[ob, dh], out_hbm.at[g // K, dh, nh], sem_o.at[ob])

        idx_copy(0, 0).start()
        idx_copy(1, 1).start()
        idx_copy(0, 0).wait()
        gather(0).start()

        @pl.loop(0, N_CHUNKS, step=2)
        def group(g):
            for sl in range(2):
                i = g + sl
                s1 = (sl + 1) % 2

                @pl.when(i + 1 < N_CHUNKS)
                def _():
                    idx_copy(i + 1, s1).wait()
                    gather(s1).start()

                gather(sl).wait()

                for ob in range(2):
                    def wait_prev():
                        for dh in range(4):
                            store(i, ob, dh).wait()

                    if sl == 0:
                        @pl.when(i >= 1)
                        def _():
                            wait_prev()
                    else:
                        wait_prev()

                    for v in range(8):
                        row0 = iota + (ob * 128 + v * 16)
                        for d in range(32):
                            vals = plsc.load_gather(
                                rows_v,
                                [jnp.broadcast_to(jnp.int32(sl), (16,)),
                                 row0,
                                 jnp.broadcast_to(jnp.int32(d), (16,))],
                            )
                            out_v[ob, d // 8, d % 8, pl.ds(v * 16, 16)] = vals

                    for dh in range(4):
                        store(i, ob, dh).start()

                @pl.when(i + 2 < N_CHUNKS)
                def _():
                    idx_copy(i + 2, sl).start()

        for ob in range(2):
            for dh in range(4):
                pltpu.make_async_copy(
                    out_v.at[ob, dh], out_hbm.at[0, dh, 0], sem_o.at[ob]).wait()

    return k(x3, table)


def kernel(x, table):
    x3 = jnp.transpose(x).astype(jnp.int32).reshape(S, K, CHUNK)
    out5 = _sc_gather(x3, table)
    out = out5.transpose(2, 4, 0, 1, 3).reshape(N, S, DIM)
    return out


# speedup vs baseline: 1.4751x; 1.4751x over previous
import functools

import jax
import jax.numpy as jnp
from jax import lax
from jax.experimental import pallas as pl
from jax.experimental.pallas import tpu as pltpu
from jax.experimental.pallas import tpu_sc as plsc

DIM = 32
NUM_CORES = 2
NUM_SUBCORES = 16
NW = NUM_CORES * NUM_SUBCORES

S = 200
N = 16384
NB = N // 128
CHUNK = 256
K = N // CHUNK
N_CHUNKS = (S * K) // NW


def _sc_gather(x3, table):
    mesh = plsc.VectorSubcoreMesh(core_axis_name="c", subcore_axis_name="s")

    @functools.partial(
        pl.kernel,
        mesh=mesh,
        out_type=jax.ShapeDtypeStruct((S, 4, NB, 8, 128), jnp.float32),
        scratch_types=[
            pltpu.VMEM((2, CHUNK), jnp.int32),
            pltpu.VMEM((2, CHUNK, DIM), jnp.float32),
            pltpu.VMEM((2, 4, 8, 128), jnp.float32),
            pltpu.SemaphoreType.DMA((2,)),
            pltpu.SemaphoreType.DMA((2,)),
            pltpu.SemaphoreType.DMA((2,)),
        ],
        compiler_params=pltpu.CompilerParams(
            use_tc_tiling_on_sc=False, needs_layout_passes=False),
    )
    def k(x_hbm, table_hbm, out_hbm, idx_v, rows_v, out_v,
          sem_i, sem_g, sem_o):
        wid = lax.axis_index("s") * NUM_CORES + lax.axis_index("c")
        iota = lax.iota(jnp.int32, 16)
        base = wid * N_CHUNKS

        def idx_copy(i, sl):
            g = base + i
            return pltpu.make_async_copy(
                x_hbm.at[g // K, g % K], idx_v.at[sl], sem_i.at[sl])

        def gather(sl):
            return pltpu.make_async_copy(
                table_hbm.at[idx_v.at[sl]], rows_v.at[sl], sem_g.at[sl])

        def store(i, ob, dh):
            g = base + i
            nh = (g % K) * 2 + ob
            return pltpu.make_async_copy(
                out_v.at[ob, dh], out_hbm.at[g // K, dh, nh], sem_o.at[ob])

        idx_copy(0, 0).start()
        idx_copy(1, 1).start()
        idx_copy(0, 0).wait()
        gather(0).start()

        @pl.loop(0, N_CHUNKS, step=2)
        def group(g):
            for sl in range(2):
                i = g + sl
                s1 = (sl + 1) % 2

                @pl.when(i + 1 < N_CHUNKS)
                def _():
                    idx_copy(i + 1, s1).wait()
                    gather(s1).start()

                gather(sl).wait()

                for ob in range(2):
                    def wait_prev():
                        for dh in range(4):
                            store(i, ob, dh).wait()

                    if sl == 0:
                        @pl.when(i >= 1)
                        def _():
                            wait_prev()
                    else:
                        wait_prev()

                    for v in range(8):
                        row0 = iota + (ob * 128 + v * 16)
                        for db in range(4):
                            vals = [
                                plsc.load_gather(
                                    rows_v,
                                    [jnp.broadcast_to(jnp.int32(sl), (16,)),
                                     row0,
                                     jnp.broadcast_to(jnp.int32(d), (16,))],
                                )
                                for d in range(db * 8, db * 8 + 8)
                            ]
                            for j, d in enumerate(range(db * 8, db * 8 + 8)):
                                out_v[ob, d // 8, d % 8,
                                      pl.ds(v * 16, 16)] = vals[j]

                    for dh in range(4):
                        store(i, ob, dh).start()

                @pl.when(i + 2 < N_CHUNKS)
                def _():
                    idx_copy(i + 2, sl).start()

        for ob in range(2):
            for dh in range(4):
                pltpu.make_async_copy(
                    out_v.at[ob, dh], out_hbm.at[0, dh, 0], sem_o.at[ob]).wait()

    return k(x3, table)


def kernel(x, table):
    x3 = jnp.transpose(x).astype(jnp.int32).reshape(S, K, CHUNK)
    out5 = _sc_gather(x3, table)
    out = out5.transpose(2, 4, 0, 1, 3).reshape(N, S, DIM)
    return out


# speedup vs baseline: 1.6316x; 1.1061x over previous
import functools

import jax
import jax.numpy as jnp
from jax import lax
from jax.experimental import pallas as pl
from jax.experimental.pallas import tpu as pltpu
from jax.experimental.pallas import tpu_sc as plsc

DIM = 32
NUM_CORES = 2
NUM_SUBCORES = 16
NW = NUM_CORES * NUM_SUBCORES

S = 200
N = 16384
NB = N // 128
CHUNK = 1024
BLOCKS = CHUNK // 128
K = N // CHUNK
N_CHUNKS = (S * K) // NW


def _sc_gather(x3, table):
    mesh = plsc.VectorSubcoreMesh(core_axis_name="c", subcore_axis_name="s")

    @functools.partial(
        pl.kernel,
        mesh=mesh,
        out_type=jax.ShapeDtypeStruct((S, 4, NB, 8, 128), jnp.float32),
        scratch_types=[
            pltpu.VMEM((2, CHUNK), jnp.int32),
            pltpu.VMEM((2, CHUNK, DIM), jnp.float32),
            pltpu.VMEM((2, 4, 8, 128), jnp.float32),
            pltpu.SemaphoreType.DMA,
            pltpu.SemaphoreType.DMA,
            pltpu.SemaphoreType.DMA,
            pltpu.SemaphoreType.DMA,
        ],
        compiler_params=pltpu.CompilerParams(
            use_tc_tiling_on_sc=False, needs_layout_passes=False),
    )
    def k(x_hbm, table_hbm, out_hbm, idx_v, rows_v, out_v,
          sem_i, sem_g, sem_o0, sem_o1):
        wid = lax.axis_index("s") * NUM_CORES + lax.axis_index("c")
        iota = lax.iota(jnp.int32, 16)
        base = wid * N_CHUNKS
        sem_o = [sem_o0, sem_o1]

        def idx_copy(i):
            g = base + i
            return pltpu.make_async_copy(
                x_hbm.at[g // K, g % K], idx_v.at[i % 2], sem_i)

        def gather(i):
            sl = i % 2
            return pltpu.make_async_copy(
                table_hbm.at[idx_v.at[sl]], rows_v.at[sl], sem_g)

        def store(i, ob, dh):
            g = base + i
            nh = (g % K) * BLOCKS + ob
            return pltpu.make_async_copy(
                out_v.at[ob % 2, dh], out_hbm.at[g // K, dh, nh],
                sem_o[ob % 2])

        idx_copy(0).start()
        idx_copy(0).wait()
        gather(0).start()
        idx_copy(1).start()

        @pl.loop(0, N_CHUNKS)
        def chunk_loop(i):
            sl = i % 2
            gather(i).wait()

            @pl.when(i + 1 < N_CHUNKS)
            def _():
                idx_copy(i + 1).wait()
                gather(i + 1).start()

            @pl.when(i + 2 < N_CHUNKS)
            def _():
                idx_copy(i + 2).start()

            chunk_sel = jnp.broadcast_to(sl, (16,))

            for ob in range(BLOCKS):
                def wait_prev():
                    for dh in range(4):
                        store(i, ob, dh).wait()

                if ob >= 2:
                    wait_prev()
                else:
                    @pl.when(i >= 1)
                    def _():
                        wait_prev()

                for v in range(8):
                    row0 = iota + (ob * 128 + v * 16)
                    for db in range(4):
                        vals = [
                            plsc.load_gather(
                                rows_v,
                                [chunk_sel, row0,
                                 jnp.broadcast_to(jnp.int32(d), (16,))],
                            )
                            for d in range(db * 8, db * 8 + 8)
                        ]
                        for j, d in enumerate(range(db * 8, db * 8 + 8)):
                            out_v[ob % 2, d // 8, d % 8,
                                  pl.ds(v * 16, 16)] = vals[j]

                for dh in range(4):
                    store(i, ob, dh).start()

        for p in range(2):
            for dh in range(4):
                pltpu.make_async_copy(
                    out_v.at[p, dh], out_hbm.at[0, dh, 0], sem_o[p]).wait()

    return k(x3, table)


def kernel(x, table):
    x3 = jnp.transpose(x).astype(jnp.int32).reshape(S, K, CHUNK)
    out5 = _sc_gather(x3, table)
    out = out5.transpose(2, 4, 0, 1, 3).reshape(N, S, DIM)
    return out
